# baseline (speedup 1.0000x reference)
import jax
import jax.numpy as jnp
from jax import lax
from jax.experimental import pallas as pl
from jax.experimental.pallas import tpu as pltpu
from jax.experimental.pallas import tpu_sc as plsc

_B = 16384
_F = 64
_FP = 128
_MEAN = 3.5
_NC = 2
_NS = 16
_NW = _NC * _NS
_BPW = _B // _NW
_CHUNK = 128
_NCHUNK = _BPW // _CHUNK
_HALF = _BPW // 2


def _body(uids_ref, iids_ref, uf_ref, if_ref, ub_ref, ib_ref, out_ref,
          uid_v, iid_v, p_v, q_v, bu_v, bi_v, out_v, sem):
    wid = lax.axis_index("s") * _NC + lax.axis_index("c")

    pltpu.sync_copy(uids_ref.at[pl.ds(wid * _NCHUNK, _NCHUNK), :], uid_v)
    pltpu.sync_copy(iids_ref.at[pl.ds(wid * _NCHUNK, _NCHUNK), :], iid_v)

    bias_copies = []
    for j in range(_NCHUNK):
        sl = pl.ds(j * _CHUNK, _CHUNK)
        bias_copies.append(
            pltpu.async_copy(ub_ref.at[uid_v.at[j]], bu_v.at[sl], sem))
        bias_copies.append(
            pltpu.async_copy(ib_ref.at[iid_v.at[j]], bi_v.at[sl], sem))

    def fire_half(h):
        cps = []
        for jj in range(_HALF // _CHUNK):
            j = h * (_HALF // _CHUNK) + jj
            sl = pl.ds(jj * _CHUNK, _CHUNK)
            cps.append(pltpu.async_copy(uf_ref.at[uid_v.at[j]], p_v.at[sl, :], sem))
            cps.append(pltpu.async_copy(if_ref.at[iid_v.at[j]], q_v.at[sl, :], sem))
        return cps

    def compute_half(h):
        def chunk16(c, carry):
            base = c * 16
            gbase = h * _HALF + base
            rows = base + lax.iota(jnp.int32, 16)
            acc = bu_v[pl.ds(gbase, 16)] + bi_v[pl.ds(gbase, 16)] + _MEAN
            for col in range(_F):
                cols = jnp.full((16,), col, jnp.int32)
                pv = plsc.load_gather(p_v, [rows, cols])
                qv = plsc.load_gather(q_v, [rows, cols])
                acc = acc + pv * qv
            out_v[pl.ds(gbase, 16)] = acc
            return carry
        lax.fori_loop(0, _HALF // 16, chunk16, 0)

    for c in fire_half(0):
        c.wait()
    for c in bias_copies:
        c.wait()
    compute_half(0)
    for c in fire_half(1):
        c.wait()
    compute_half(1)

    pltpu.sync_copy(out_v, out_ref.at[pl.ds(wid * _BPW, _BPW)])


@jax.jit
def _sc_call(uids2, iids2, uf, itf, ub, ib):
    mesh = plsc.VectorSubcoreMesh(core_axis_name="c", subcore_axis_name="s")
    return pl.kernel(
        _body,
        out_type=jax.ShapeDtypeStruct((_B,), jnp.float32),
        mesh=mesh,
        compiler_params=pltpu.CompilerParams(
            needs_layout_passes=False, use_tc_tiling_on_sc=False),
        scratch_types=[
            pltpu.VMEM((_NCHUNK, _CHUNK), jnp.int32),
            pltpu.VMEM((_NCHUNK, _CHUNK), jnp.int32),
            pltpu.VMEM((_HALF, _FP), jnp.float32),
            pltpu.VMEM((_HALF, _FP), jnp.float32),
            pltpu.VMEM((_BPW,), jnp.float32),
            pltpu.VMEM((_BPW,), jnp.float32),
            pltpu.VMEM((_BPW,), jnp.float32),
            pltpu.SemaphoreType.DMA,
        ],
    )(uids2, iids2, uf, itf, ub, ib)


def kernel(user_ids, item_ids, user_factors, item_factors, user_bias, item_bias):
    uids2 = user_ids.astype(jnp.int32).reshape(_B // _CHUNK, _CHUNK)
    iids2 = item_ids.astype(jnp.int32).reshape(_B // _CHUNK, _CHUNK)
    uf = jnp.pad(user_factors, ((0, 0), (0, _FP - _F)))
    itf = jnp.pad(item_factors, ((0, 0), (0, _FP - _F)))
    ub = user_bias.reshape(-1)
    ib = item_bias.reshape(-1)
    return _sc_call(uids2, iids2, uf, itf, ub, ib)


# speedup vs baseline: 1.4633x; 1.4633x over previous
import jax
import jax.numpy as jnp
from jax import lax
from jax.experimental import pallas as pl
from jax.experimental.pallas import tpu as pltpu
from jax.experimental.pallas import tpu_sc as plsc

_B = 16384
_F = 64
_MEAN = 3.5
_NC = 2
_NS = 16
_NW = _NC * _NS
_BPW = _B // _NW
_CHUNK = 128
_NCHUNK = _BPW // _CHUNK
_NU = 1000000
_NI = 100000
_UCOLS = (_NU + 127) // 128
_ICOLS = (_NI + 127) // 128
_UPW = (_UCOLS + _NW - 1) // _NW
_IPW = (_ICOLS + _NW - 1) // _NW
_PMAX = 128
_PROWS = _B + _NW


def _scan_table(tab_ref, tail_ref, ids_ref, out_ref, wid, ncols, perw,
                ulist, mu, mb, slab0, slab1, pstage, bstage,
                s0, s1, psem):
    c0 = wid * perw
    c1 = jnp.minimum(c0 + perw, ncols)
    c0 = jnp.minimum(c0, ncols)

    pltpu.sync_copy(ids_ref, ulist)

    def filt(k, cnt):
        uvec = ulist[pl.ds(k * 16, 16)]
        bvec = k * 16 + lax.iota(jnp.int32, 16)
        tc = uvec >> 7
        m = (tc >= c0) & (tc < c1)
        plsc.store_compressed(mu.at[pl.ds(cnt, 16)], uvec, mask=m)
        plsc.store_compressed(mb.at[pl.ds(cnt, 16)], bvec, mask=m)
        return cnt + jnp.sum(m.astype(jnp.int32))
    kw = lax.fori_loop(0, _B // 16, filt, jnp.int32(0))

    dummy = _B + wid
    for r in range(_PMAX // 16):
        bstage[pl.ds(r * 16, 16)] = jnp.full((16,), dummy, jnp.int32)

    def fire(c, buf, sem):
        @pl.when(c < ncols - 1)
        def _():
            off = pl.multiple_of(c * 128, 128)
            pltpu.async_copy(tab_ref.at[:, :, pl.ds(off, 128)], buf, sem)

        @pl.when(c >= ncols - 1)
        def _():
            pltpu.async_copy(tail_ref, buf, sem)

    def process(c, buf, state):
        pcnt = state

        def match_chunk(j, st):
            pc = st
            uvec = mu[pl.ds(j * 16, 16)]
            bvec = mb[pl.ds(j * 16, 16)]
            lanes = j * 16 + lax.iota(jnp.int32, 16)
            m = ((uvec >> 7) == c) & (lanes < kw)
            nm = jnp.sum(m.astype(jnp.int32))

            def have(pc_in):
                plsc.store_compressed(mu.at[pl.ds(_B, 16)], uvec, mask=m)
                plsc.store_compressed(mb.at[pl.ds(_B, 16)], bvec, mask=m)

                def one(n, pc2):
                    u = mu[pl.ds(_B + n, 16)][0]
                    b = mb[pl.ds(_B + n, 16)][0]
                    col = u & 127
                    for k in range(_F // 16):
                        fvec = k * 16 + lax.iota(jnp.int32, 16)
                        a = fvec >> 3
                        s = fvec & 7
                        cs = jnp.full((16,), col, jnp.int32)
                        vals = plsc.load_gather(buf, [a, s, cs])
                        pstage[pc2, pl.ds(k * 16, 16)] = vals
                    plsc.store_scatter(
                        bstage, [jnp.full((16,), pc2, jnp.int32)],
                        jnp.full((16,), b, jnp.int32),
                        mask=lax.iota(jnp.int32, 16) == 0)
                    pc2 = pc2 + 1

                    @pl.when(pc2 >= _PMAX)
                    def _():
                        pltpu.async_copy(pstage, out_ref.at[bstage], psem).wait()
                        for r in range(_PMAX // 16):
                            bstage[pl.ds(r * 16, 16)] = jnp.full(
                                (16,), dummy, jnp.int32)
                    return jnp.where(pc2 >= _PMAX, 0, pc2)
                return lax.fori_loop(0, nm, one, pc_in)

            pc = lax.cond(nm > 0, have, lambda x: x, pc)
            return pc
        return lax.fori_loop(0, (kw + 15) // 16, match_chunk, pcnt)

    n = c1 - c0

    @pl.when(n > 0)
    def _():
        fire(c0, slab0, s0)

    def pair(t, pcnt):
        c = c0 + 2 * t

        @pl.when(c + 1 < c1)
        def _():
            fire(c + 1, slab1, s1)
        pltpu.make_async_copy(
            tab_ref.at[:, :, pl.ds(0, 128)], slab0, s0).wait()
        pcnt = process(c, slab0, pcnt)

        @pl.when(c + 2 < c1)
        def _():
            fire(c + 2, slab0, s0)

        def odd(pc):
            pltpu.make_async_copy(
                tab_ref.at[:, :, pl.ds(0, 128)], slab1, s1).wait()
            return process(c + 1, slab1, pc)
        pcnt = lax.cond(c + 1 < c1, odd, lambda x: x, pcnt)
        return pcnt

    pcnt = lax.fori_loop(0, (n + 1) // 2, pair, jnp.int32(0))

    @pl.when(pcnt > 0)
    def _():
        pltpu.async_copy(pstage, out_ref.at[bstage], psem).wait()


def _k1_body(uids_ref, iids_ref, uf_ref, if_ref, utail_ref, itail_ref,
             p_out, q_out,
             ulist, mu, mb, slab0, slab1, pstage, bstage, s0, s1, psem):
    wid = lax.axis_index("s") * _NC + lax.axis_index("c")
    _scan_table(uf_ref, utail_ref, uids_ref, p_out, wid, _UCOLS, _UPW,
                ulist, mu, mb, slab0, slab1, pstage, bstage, s0, s1, psem)
    _scan_table(if_ref, itail_ref, iids_ref, q_out, wid, _ICOLS, _IPW,
                ulist, mu, mb, slab0, slab1, pstage, bstage, s0, s1, psem)


@jax.jit
def _k1(uids, iids, uf4, if4, utail, itail):
    mesh = plsc.VectorSubcoreMesh(core_axis_name="c", subcore_axis_name="s")
    return pl.kernel(
        _k1_body,
        out_type=(
            jax.ShapeDtypeStruct((_PROWS, 128), jnp.float32),
            jax.ShapeDtypeStruct((_PROWS, 128), jnp.float32),
        ),
        mesh=mesh,
        compiler_params=pltpu.CompilerParams(
            needs_layout_passes=False, use_tc_tiling_on_sc=True),
        scratch_types=[
            pltpu.VMEM((_B,), jnp.int32),
            pltpu.VMEM((_B + 32,), jnp.int32),
            pltpu.VMEM((_B + 32,), jnp.int32),
            pltpu.VMEM((8, 8, 128), jnp.float32),
            pltpu.VMEM((8, 8, 128), jnp.float32),
            pltpu.VMEM((_PMAX, 128), jnp.float32),
            pltpu.VMEM((_PMAX,), jnp.int32),
            pltpu.SemaphoreType.DMA,
            pltpu.SemaphoreType.DMA,
            pltpu.SemaphoreType.DMA,
        ],
    )(uids, iids, uf4, if4, utail, itail)


def _k2_body(uids_ref, iids_ref, p_ref, q_ref, ub_ref, ib_ref, out_ref,
             uid_v, iid_v, p_v, q_v, bu_v, bi_v, out_v, sem):
    wid = lax.axis_index("s") * _NC + lax.axis_index("c")

    pltpu.sync_copy(uids_ref.at[pl.ds(wid * _NCHUNK, _NCHUNK), :], uid_v)
    pltpu.sync_copy(iids_ref.at[pl.ds(wid * _NCHUNK, _NCHUNK), :], iid_v)

    bias_copies = []
    for j in range(_NCHUNK):
        sl = pl.ds(j * _CHUNK, _CHUNK)
        bias_copies.append(
            pltpu.async_copy(ub_ref.at[uid_v.at[j]], bu_v.at[sl], sem))
        bias_copies.append(
            pltpu.async_copy(ib_ref.at[iid_v.at[j]], bi_v.at[sl], sem))
    for cp in bias_copies:
        cp.wait()

    for h in range(2):
        base_row = wid * _BPW + h * 256
        pltpu.sync_copy(p_ref.at[pl.ds(base_row, 256), :], p_v)
        pltpu.sync_copy(q_ref.at[pl.ds(base_row, 256), :], q_v)

        def chunk16(c, carry):
            rows = c * 16 + lax.iota(jnp.int32, 16)
            gbase = h * 256 + c * 16
            acc = bu_v[pl.ds(gbase, 16)] + bi_v[pl.ds(gbase, 16)] + _MEAN
            for f in range(_F):
                fs = jnp.full((16,), f, jnp.int32)
                pv = plsc.load_gather(p_v, [rows, fs])
                qv = plsc.load_gather(q_v, [rows, fs])
                acc = acc + pv * qv
            out_v[pl.ds(gbase, 16)] = acc
            return carry
        lax.fori_loop(0, 16, chunk16, 0)

    pltpu.sync_copy(out_v, out_ref.at[pl.ds(wid * _BPW, _BPW)])


@jax.jit
def _k2(uids2, iids2, p, q, ub, ib):
    mesh = plsc.VectorSubcoreMesh(core_axis_name="c", subcore_axis_name="s")
    return pl.kernel(
        _k2_body,
        out_type=jax.ShapeDtypeStruct((_B,), jnp.float32),
        mesh=mesh,
        compiler_params=pltpu.CompilerParams(
            needs_layout_passes=False, use_tc_tiling_on_sc=False),
        scratch_types=[
            pltpu.VMEM((_NCHUNK, _CHUNK), jnp.int32),
            pltpu.VMEM((_NCHUNK, _CHUNK), jnp.int32),
            pltpu.VMEM((256, 128), jnp.float32),
            pltpu.VMEM((256, 128), jnp.float32),
            pltpu.VMEM((_BPW,), jnp.float32),
            pltpu.VMEM((_BPW,), jnp.float32),
            pltpu.VMEM((_BPW,), jnp.float32),
            pltpu.SemaphoreType.DMA,
        ],
    )(uids2, iids2, p, q, ub, ib)


def _tail3(tab, n):
    start = (n // 128) * 128
    t = tab[start:].T
    t = jnp.pad(t, ((0, 0), (0, 128 - (n - start))))
    return t.reshape(8, 8, 128)


def kernel(user_ids, item_ids, user_factors, item_factors, user_bias, item_bias):
    uids = user_ids.astype(jnp.int32)
    iids = item_ids.astype(jnp.int32)
    uids2 = uids.reshape(_B // _CHUNK, _CHUNK)
    iids2 = iids.reshape(_B // _CHUNK, _CHUNK)
    uf4 = user_factors.T.reshape(8, 8, _NU)
    if4 = item_factors.T.reshape(8, 8, _NI)
    utail = _tail3(user_factors, _NU)
    itail = _tail3(item_factors, _NI)
    ub = user_bias.reshape(-1)
    ib = item_bias.reshape(-1)
    p, q = _k1(uids, iids, uf4, if4, utail, itail)
    return _k2(uids2, iids2, p, q, ub, ib)


# speedup vs baseline: 1.9007x; 1.2989x over previous
import jax
import jax.numpy as jnp
from jax import lax
from jax.experimental import pallas as pl
from jax.experimental.pallas import tpu as pltpu
from jax.experimental.pallas import tpu_sc as plsc

_B = 16384
_F = 64
_MEAN = 3.5
_NC = 2
_NS = 16
_NW = _NC * _NS
_BPW = _B // _NW
_CHUNK = 128
_NCHUNK = _BPW // _CHUNK
_NU = 1000000
_NI = 100000
_UCOLS = (_NU + 127) // 128
_ICOLS = (_NI + 127) // 128
_UPW = (_UCOLS + _NW - 1) // _NW
_IPW = (_ICOLS + _NW - 1) // _NW
_PMAX = 128
_HBUF = 272
_PROWS = _B + _NW


def _scan_table(tab_ref, tail_ref, ids_ref, out_ref, wid, ncols, perw,
                ulist, mu, mb, musort, mbsort, hist, offs, cur,
                slab0, slab1, pstage, bstage, s0, s1, psem):
    lane0 = lax.iota(jnp.int32, 16) == 0
    c0 = wid * perw
    c1 = jnp.minimum(c0 + perw, ncols)
    c0 = jnp.minimum(c0, ncols)

    pltpu.sync_copy(ids_ref, ulist)

    def filt(k, cnt):
        uvec = ulist[pl.ds(k * 16, 16)]
        bvec = k * 16 + lax.iota(jnp.int32, 16)
        tc = uvec >> 7
        m = (tc >= c0) & (tc < c1)
        plsc.store_compressed(mu.at[pl.ds(cnt, 16)], uvec, mask=m)
        plsc.store_compressed(mb.at[pl.ds(cnt, 16)], bvec, mask=m)
        return cnt + plsc.all_reduce_population_count(m)[0]
    kw = lax.fori_loop(0, _B // 16, filt, jnp.int32(0))

    for r in range(_HBUF // 16):
        hist[pl.ds(r * 16, 16)] = jnp.zeros((16,), jnp.int32)

    ones = jnp.ones((16,), jnp.int32)
    lanes16 = lax.iota(jnp.int32, 16)

    def hpass(k, carry):
        uvec = mu[pl.ds(k * 16, 16)]
        m = (k * 16 + lanes16) < kw
        rel = (uvec >> 7) - c0
        rel = jnp.where(m, rel, _HBUF - 1)
        plsc.addupdate_scatter(hist, [rel], ones, mask=m)
        return carry
    lax.fori_loop(0, (kw + 15) // 16, hpass, 0)

    def pref(r, carry):
        v = hist[pl.ds(r * 16, 16)]
        cs = plsc.cumsum(v)
        ex = cs - v + carry
        offs[pl.ds(r * 16, 16)] = ex
        cur[pl.ds(r * 16, 16)] = ex
        return carry + cs[15]
    lax.fori_loop(0, _HBUF // 16, pref, jnp.int32(0))

    def place(n, carry):
        u = mu[pl.ds(n, 16)][0]
        b = mb[pl.ds(n, 16)][0]
        rel = (u >> 7) - c0
        pos = cur[pl.ds(rel, 16)][0]
        plsc.store_scatter(musort, [jnp.full((16,), pos, jnp.int32)],
                           jnp.full((16,), u, jnp.int32), mask=lane0)
        plsc.store_scatter(mbsort, [jnp.full((16,), pos, jnp.int32)],
                           jnp.full((16,), b, jnp.int32), mask=lane0)
        plsc.store_scatter(cur, [jnp.full((16,), rel, jnp.int32)],
                           jnp.full((16,), pos + 1, jnp.int32), mask=lane0)
        return carry
    lax.fori_loop(0, kw, place, 0)

    dummy = _B + wid
    for r in range(_PMAX // 16):
        bstage[pl.ds(r * 16, 16)] = jnp.full((16,), dummy, jnp.int32)

    def fire(c, buf, sem):
        @pl.when(c < ncols - 1)
        def _():
            off = pl.multiple_of(c * 128, 128)
            pltpu.async_copy(tab_ref.at[:, :, pl.ds(off, 128)], buf, sem)

        @pl.when(c >= ncols - 1)
        def _():
            pltpu.async_copy(tail_ref, buf, sem)

    def process(c, buf, state):
        rel = c - c0
        lo = offs[pl.ds(rel, 16)][0]
        hi = offs[pl.ds(rel + 1, 16)][0]

        def one(n, pc2):
            u = musort[pl.ds(lo + n, 16)][0]
            b = mbsort[pl.ds(lo + n, 16)][0]
            col = u & 127
            for k in range(_F // 16):
                fvec = k * 16 + lax.iota(jnp.int32, 16)
                a = fvec >> 3
                s = fvec & 7
                cs = jnp.full((16,), col, jnp.int32)
                vals = plsc.load_gather(buf, [a, s, cs])
                pstage[pc2, pl.ds(k * 16, 16)] = vals
            plsc.store_scatter(
                bstage, [jnp.full((16,), pc2, jnp.int32)],
                jnp.full((16,), b, jnp.int32), mask=lane0)
            pc2 = pc2 + 1

            @pl.when(pc2 >= _PMAX)
            def _():
                pltpu.async_copy(pstage, out_ref.at[bstage], psem).wait()
                for r in range(_PMAX // 16):
                    bstage[pl.ds(r * 16, 16)] = jnp.full(
                        (16,), dummy, jnp.int32)
            return jnp.where(pc2 >= _PMAX, 0, pc2)
        return lax.fori_loop(0, hi - lo, one, state)

    n = c1 - c0

    @pl.when(n > 0)
    def _():
        fire(c0, slab0, s0)

    def pair(t, pcnt):
        c = c0 + 2 * t

        @pl.when(c + 1 < c1)
        def _():
            fire(c + 1, slab1, s1)
        pltpu.make_async_copy(
            tab_ref.at[:, :, pl.ds(0, 128)], slab0, s0).wait()
        pcnt = process(c, slab0, pcnt)

        @pl.when(c + 2 < c1)
        def _():
            fire(c + 2, slab0, s0)

        def odd(pc):
            pltpu.make_async_copy(
                tab_ref.at[:, :, pl.ds(0, 128)], slab1, s1).wait()
            return process(c + 1, slab1, pc)
        pcnt = lax.cond(c + 1 < c1, odd, lambda x: x, pcnt)
        return pcnt

    pcnt = lax.fori_loop(0, (n + 1) // 2, pair, jnp.int32(0))

    @pl.when(pcnt > 0)
    def _():
        pltpu.async_copy(pstage, out_ref.at[bstage], psem).wait()


def _k1_body(uids_ref, iids_ref, uf_ref, if_ref, utail_ref, itail_ref,
             p_out, q_out,
             ulist, mu, mb, musort, mbsort, hist, offs, cur,
             slab0, slab1, pstage, bstage, s0, s1, psem):
    wid = lax.axis_index("s") * _NC + lax.axis_index("c")
    _scan_table(uf_ref, utail_ref, uids_ref, p_out, wid, _UCOLS, _UPW,
                ulist, mu, mb, musort, mbsort, hist, offs, cur,
                slab0, slab1, pstage, bstage, s0, s1, psem)
    _scan_table(if_ref, itail_ref, iids_ref, q_out, wid, _ICOLS, _IPW,
                ulist, mu, mb, musort, mbsort, hist, offs, cur,
                slab0, slab1, pstage, bstage, s0, s1, psem)


@jax.jit
def _k1(uids, iids, uf4, if4, utail, itail):
    mesh = plsc.VectorSubcoreMesh(core_axis_name="c", subcore_axis_name="s")
    return pl.kernel(
        _k1_body,
        out_type=(
            jax.ShapeDtypeStruct((_PROWS, 128), jnp.float32),
            jax.ShapeDtypeStruct((_PROWS, 128), jnp.float32),
        ),
        mesh=mesh,
        compiler_params=pltpu.CompilerParams(
            needs_layout_passes=False, use_tc_tiling_on_sc=True),
        scratch_types=[
            pltpu.VMEM((_B,), jnp.int32),
            pltpu.VMEM((_B + 32,), jnp.int32),
            pltpu.VMEM((_B + 32,), jnp.int32),
            pltpu.VMEM((_B + 32,), jnp.int32),
            pltpu.VMEM((_B + 32,), jnp.int32),
            pltpu.VMEM((_HBUF,), jnp.int32),
            pltpu.VMEM((_HBUF + 16,), jnp.int32),
            pltpu.VMEM((_HBUF + 16,), jnp.int32),
            pltpu.VMEM((8, 8, 128), jnp.float32),
            pltpu.VMEM((8, 8, 128), jnp.float32),
            pltpu.VMEM((_PMAX, 128), jnp.float32),
            pltpu.VMEM((_PMAX,), jnp.int32),
            pltpu.SemaphoreType.DMA,
            pltpu.SemaphoreType.DMA,
            pltpu.SemaphoreType.DMA,
        ],
    )(uids, iids, uf4, if4, utail, itail)


def _k2_body(uids_ref, iids_ref, p_ref, q_ref, ub_ref, ib_ref, out_ref,
             uid_v, iid_v, p_v, q_v, bu_v, bi_v, out_v, sem):
    wid = lax.axis_index("s") * _NC + lax.axis_index("c")

    pltpu.sync_copy(uids_ref.at[pl.ds(wid * _NCHUNK, _NCHUNK), :], uid_v)
    pltpu.sync_copy(iids_ref.at[pl.ds(wid * _NCHUNK, _NCHUNK), :], iid_v)

    bias_copies = []
    for j in range(_NCHUNK):
        sl = pl.ds(j * _CHUNK, _CHUNK)
        bias_copies.append(
            pltpu.async_copy(ub_ref.at[uid_v.at[j]], bu_v.at[sl], sem))
        bias_copies.append(
            pltpu.async_copy(ib_ref.at[iid_v.at[j]], bi_v.at[sl], sem))
    for cp in bias_copies:
        cp.wait()

    for h in range(2):
        base_row = wid * _BPW + h * 256
        pltpu.sync_copy(p_ref.at[pl.ds(base_row, 256), :], p_v)
        pltpu.sync_copy(q_ref.at[pl.ds(base_row, 256), :], q_v)

        def chunk16(c, carry):
            rows = c * 16 + lax.iota(jnp.int32, 16)
            gbase = h * 256 + c * 16
            acc = bu_v[pl.ds(gbase, 16)] + bi_v[pl.ds(gbase, 16)] + _MEAN
            for f in range(_F):
                fs = jnp.full((16,), f, jnp.int32)
                pv = plsc.load_gather(p_v, [rows, fs])
                qv = plsc.load_gather(q_v, [rows, fs])
                acc = acc + pv * qv
            out_v[pl.ds(gbase, 16)] = acc
            return carry
        lax.fori_loop(0, 16, chunk16, 0)

    pltpu.sync_copy(out_v, out_ref.at[pl.ds(wid * _BPW, _BPW)])


@jax.jit
def _k2(uids2, iids2, p, q, ub, ib):
    mesh = plsc.VectorSubcoreMesh(core_axis_name="c", subcore_axis_name="s")
    return pl.kernel(
        _k2_body,
        out_type=jax.ShapeDtypeStruct((_B,), jnp.float32),
        mesh=mesh,
        compiler_params=pltpu.CompilerParams(
            needs_layout_passes=False, use_tc_tiling_on_sc=False),
        scratch_types=[
            pltpu.VMEM((_NCHUNK, _CHUNK), jnp.int32),
            pltpu.VMEM((_NCHUNK, _CHUNK), jnp.int32),
            pltpu.VMEM((256, 128), jnp.float32),
            pltpu.VMEM((256, 128), jnp.float32),
            pltpu.VMEM((_BPW,), jnp.float32),
            pltpu.VMEM((_BPW,), jnp.float32),
            pltpu.VMEM((_BPW,), jnp.float32),
            pltpu.SemaphoreType.DMA,
        ],
    )(uids2, iids2, p, q, ub, ib)


def _tail3(tab, n):
    start = (n // 128) * 128
    t = tab[start:].T
    t = jnp.pad(t, ((0, 0), (0, 128 - (n - start))))
    return t.reshape(8, 8, 128)


def kernel(user_ids, item_ids, user_factors, item_factors, user_bias, item_bias):
    uids = user_ids.astype(jnp.int32)
    iids = item_ids.astype(jnp.int32)
    uids2 = uids.reshape(_B // _CHUNK, _CHUNK)
    iids2 = iids.reshape(_B // _CHUNK, _CHUNK)
    uf4 = user_factors.T.reshape(8, 8, _NU)
    if4 = item_factors.T.reshape(8, 8, _NI)
    utail = _tail3(user_factors, _NU)
    itail = _tail3(item_factors, _NI)
    ub = user_bias.reshape(-1)
    ib = item_bias.reshape(-1)
    p, q = _k1(uids, iids, uf4, if4, utail, itail)
    return _k2(uids2, iids2, p, q, ub, ib)


# speedup vs baseline: 2.2526x; 1.1852x over previous
import jax
import jax.numpy as jnp
from jax import lax
from jax.experimental import pallas as pl
from jax.experimental.pallas import tpu as pltpu
from jax.experimental.pallas import tpu_sc as plsc

_B = 16384
_F = 64
_MEAN = 3.5
_NC = 2
_NS = 16
_NW = _NC * _NS
_BPW = _B // _NW
_CHUNK = 128
_NCHUNK = _BPW // _CHUNK
_NU = 1000000
_NI = 100000
_UCOLS = (_NU + 127) // 128
_ICOLS = (_NI + 127) // 128
_UPW = (_UCOLS + _NW - 1) // _NW
_IPW = (_ICOLS + _NW - 1) // _NW
_PMAX = 128
_HBUF = 272
_PROWS = _B + _NW


def _scan_table(tab_ref, tail_ref, ids_ref, out_ref, wid, ncols, perw,
                mu, mb, musort, mbsort, hist, offs, cur,
                slab0, slab1, slab2, slab3, pstage, bstage,
                s0, s1, s2, s3, psem):
    lane0 = lax.iota(jnp.int32, 16) == 0
    c0 = wid * perw
    c1 = jnp.minimum(c0 + perw, ncols)
    c0 = jnp.minimum(c0, ncols)

    pltpu.sync_copy(ids_ref, musort.at[pl.ds(0, _B)])

    def filt(k4, cnt):
        for e in range(4):
            k = k4 * 4 + e
            uvec = musort[pl.ds(k * 16, 16)]
            bvec = k * 16 + lax.iota(jnp.int32, 16)
            tc = uvec >> 7
            m = (tc >= c0) & (tc < c1)
            plsc.store_compressed(mu.at[pl.ds(cnt, 16)], uvec, mask=m)
            plsc.store_compressed(mb.at[pl.ds(cnt, 16)], bvec, mask=m)
            cnt = cnt + plsc.all_reduce_population_count(m)[0]
        return cnt
    kw = lax.fori_loop(0, _B // 64, filt, jnp.int32(0))

    for r in range(_HBUF // 16):
        hist[pl.ds(r * 16, 16)] = jnp.zeros((16,), jnp.int32)

    ones = jnp.ones((16,), jnp.int32)
    lanes16 = lax.iota(jnp.int32, 16)

    def hpass(k, carry):
        uvec = mu[pl.ds(k * 16, 16)]
        m = (k * 16 + lanes16) < kw
        rel = (uvec >> 7) - c0
        rel = jnp.where(m, rel, _HBUF - 1)
        plsc.addupdate_scatter(hist, [rel], ones, mask=m)
        return carry
    lax.fori_loop(0, (kw + 15) // 16, hpass, 0)

    def pref(r, carry):
        v = hist[pl.ds(r * 16, 16)]
        cs = plsc.cumsum(v)
        ex = cs - v + carry
        offs[pl.ds(r * 16, 16)] = ex
        cur[pl.ds(r * 16, 16)] = ex
        return carry + cs[15]
    lax.fori_loop(0, _HBUF // 16, pref, jnp.int32(0))

    def place(n, carry):
        u = mu[pl.ds(n, 16)][0]
        b = mb[pl.ds(n, 16)][0]
        rel = (u >> 7) - c0
        pos = cur[pl.ds(rel, 16)][0]
        plsc.store_scatter(musort, [jnp.full((16,), pos, jnp.int32)],
                           jnp.full((16,), u, jnp.int32), mask=lane0)
        plsc.store_scatter(mbsort, [jnp.full((16,), pos, jnp.int32)],
                           jnp.full((16,), b, jnp.int32), mask=lane0)
        plsc.store_scatter(cur, [jnp.full((16,), rel, jnp.int32)],
                           jnp.full((16,), pos + 1, jnp.int32), mask=lane0)
        return carry
    lax.fori_loop(0, kw, place, 0)

    dummy = _B + wid
    for r in range(_PMAX // 16):
        bstage[pl.ds(r * 16, 16)] = jnp.full((16,), dummy, jnp.int32)

    def fire(c, buf, sem):
        @pl.when(c < ncols - 1)
        def _():
            off = pl.multiple_of(c * 128, 128)
            pltpu.async_copy(tab_ref.at[:, :, pl.ds(off, 128)], buf, sem)

        @pl.when(c >= ncols - 1)
        def _():
            pltpu.async_copy(tail_ref, buf, sem)

    def process(c, buf, state):
        rel = c - c0
        lo = offs[pl.ds(rel, 16)][0]
        hi = offs[pl.ds(rel + 1, 16)][0]

        def one(n, pc2):
            u = musort[pl.ds(lo + n, 16)][0]
            b = mbsort[pl.ds(lo + n, 16)][0]
            col = u & 127
            for k in range(_F // 16):
                fvec = k * 16 + lax.iota(jnp.int32, 16)
                a = fvec >> 3
                s = fvec & 7
                cs = jnp.full((16,), col, jnp.int32)
                vals = plsc.load_gather(buf, [a, s, cs])
                pstage[pc2, pl.ds(k * 16, 16)] = vals
            plsc.store_scatter(
                bstage, [jnp.full((16,), pc2, jnp.int32)],
                jnp.full((16,), b, jnp.int32), mask=lane0)
            pc2 = pc2 + 1

            @pl.when(pc2 >= _PMAX)
            def _():
                pltpu.async_copy(pstage, out_ref.at[bstage], psem).wait()
                for r in range(_PMAX // 16):
                    bstage[pl.ds(r * 16, 16)] = jnp.full(
                        (16,), dummy, jnp.int32)
            return jnp.where(pc2 >= _PMAX, 0, pc2)
        return lax.fori_loop(0, hi - lo, one, state)

    n = c1 - c0
    slabs = (slab0, slab1, slab2, slab3)
    sems = (s0, s1, s2, s3)
    for j in range(4):
        @pl.when(c0 + j < c1)
        def _(j=j):
            fire(c0 + j, slabs[j], sems[j])

    def ring(t, pcnt):
        for j in range(4):
            c = c0 + 4 * t + j

            def go(pc, j=j, c=c):
                pltpu.make_async_copy(
                    tab_ref.at[:, :, pl.ds(0, 128)], slabs[j], sems[j]).wait()
                pc = process(c, slabs[j], pc)

                @pl.when(c + 4 < c1)
                def _():
                    fire(c + 4, slabs[j], sems[j])
                return pc
            pcnt = lax.cond(c < c1, go, lambda x: x, pcnt)
        return pcnt

    pcnt = lax.fori_loop(0, (n + 3) // 4, ring, jnp.int32(0))

    @pl.when(pcnt > 0)
    def _():
        pltpu.async_copy(pstage, out_ref.at[bstage], psem).wait()


def _k1_body(uids_ref, iids_ref, uf_ref, if_ref, utail_ref, itail_ref,
             p_out, q_out,
             mu, mb, musort, mbsort, hist, offs, cur,
             slab0, slab1, slab2, slab3, pstage, bstage,
             s0, s1, s2, s3, psem):
    wid = lax.axis_index("s") * _NC + lax.axis_index("c")
    _scan_table(uf_ref, utail_ref, uids_ref, p_out, wid, _UCOLS, _UPW,
                mu, mb, musort, mbsort, hist, offs, cur,
                slab0, slab1, slab2, slab3, pstage, bstage,
                s0, s1, s2, s3, psem)
    _scan_table(if_ref, itail_ref, iids_ref, q_out, wid, _ICOLS, _IPW,
                mu, mb, musort, mbsort, hist, offs, cur,
                slab0, slab1, slab2, slab3, pstage, bstage,
                s0, s1, s2, s3, psem)


@jax.jit
def _k1(uids, iids, uf4, if4, utail, itail):
    mesh = plsc.VectorSubcoreMesh(core_axis_name="c", subcore_axis_name="s")
    return pl.kernel(
        _k1_body,
        out_type=(
            jax.ShapeDtypeStruct((_PROWS, 128), jnp.float32),
            jax.ShapeDtypeStruct((_PROWS, 128), jnp.float32),
        ),
        mesh=mesh,
        compiler_params=pltpu.CompilerParams(
            needs_layout_passes=False, use_tc_tiling_on_sc=True),
        scratch_types=[
            pltpu.VMEM((_B + 32,), jnp.int32),
            pltpu.VMEM((_B + 32,), jnp.int32),
            pltpu.VMEM((_B + 32,), jnp.int32),
            pltpu.VMEM((_B + 32,), jnp.int32),
            pltpu.VMEM((_HBUF,), jnp.int32),
            pltpu.VMEM((_HBUF + 16,), jnp.int32),
            pltpu.VMEM((_HBUF + 16,), jnp.int32),
            pltpu.VMEM((8, 8, 128), jnp.float32),
            pltpu.VMEM((8, 8, 128), jnp.float32),
            pltpu.VMEM((8, 8, 128), jnp.float32),
            pltpu.VMEM((8, 8, 128), jnp.float32),
            pltpu.VMEM((_PMAX, 128), jnp.float32),
            pltpu.VMEM((_PMAX,), jnp.int32),
            pltpu.SemaphoreType.DMA,
            pltpu.SemaphoreType.DMA,
            pltpu.SemaphoreType.DMA,
            pltpu.SemaphoreType.DMA,
            pltpu.SemaphoreType.DMA,
        ],
    )(uids, iids, uf4, if4, utail, itail)


def _k2_body(uids_ref, iids_ref, p_ref, q_ref, ub_ref, ib_ref, out_ref,
             uid_v, iid_v, p_v, q_v, bu_v, bi_v, out_v, sem):
    wid = lax.axis_index("s") * _NC + lax.axis_index("c")

    pltpu.sync_copy(uids_ref.at[pl.ds(wid * _NCHUNK, _NCHUNK), :], uid_v)
    pltpu.sync_copy(iids_ref.at[pl.ds(wid * _NCHUNK, _NCHUNK), :], iid_v)

    bias_copies = []
    for j in range(_NCHUNK):
        sl = pl.ds(j * _CHUNK, _CHUNK)
        bias_copies.append(
            pltpu.async_copy(ub_ref.at[uid_v.at[j]], bu_v.at[sl], sem))
        bias_copies.append(
            pltpu.async_copy(ib_ref.at[iid_v.at[j]], bi_v.at[sl], sem))
    for cp in bias_copies:
        cp.wait()

    for h in range(2):
        base_row = wid * _BPW + h * 256
        pltpu.sync_copy(p_ref.at[pl.ds(base_row, 256), :], p_v)
        pltpu.sync_copy(q_ref.at[pl.ds(base_row, 256), :], q_v)

        def chunk16(c, carry):
            rows = c * 16 + lax.iota(jnp.int32, 16)
            gbase = h * 256 + c * 16
            acc = bu_v[pl.ds(gbase, 16)] + bi_v[pl.ds(gbase, 16)] + _MEAN
            for f in range(_F):
                fs = jnp.full((16,), f, jnp.int32)
                pv = plsc.load_gather(p_v, [rows, fs])
                qv = plsc.load_gather(q_v, [rows, fs])
                acc = acc + pv * qv
            out_v[pl.ds(gbase, 16)] = acc
            return carry
        lax.fori_loop(0, 16, chunk16, 0)

    pltpu.sync_copy(out_v, out_ref.at[pl.ds(wid * _BPW, _BPW)])


@jax.jit
def _k2(uids2, iids2, p, q, ub, ib):
    mesh = plsc.VectorSubcoreMesh(core_axis_name="c", subcore_axis_name="s")
    return pl.kernel(
        _k2_body,
        out_type=jax.ShapeDtypeStruct((_B,), jnp.float32),
        mesh=mesh,
        compiler_params=pltpu.CompilerParams(
            needs_layout_passes=False, use_tc_tiling_on_sc=False),
        scratch_types=[
            pltpu.VMEM((_NCHUNK, _CHUNK), jnp.int32),
            pltpu.VMEM((_NCHUNK, _CHUNK), jnp.int32),
            pltpu.VMEM((256, 128), jnp.float32),
            pltpu.VMEM((256, 128), jnp.float32),
            pltpu.VMEM((_BPW,), jnp.float32),
            pltpu.VMEM((_BPW,), jnp.float32),
            pltpu.VMEM((_BPW,), jnp.float32),
            pltpu.SemaphoreType.DMA,
        ],
    )(uids2, iids2, p, q, ub, ib)


def _tail3(tab, n):
    start = (n // 128) * 128
    t = tab[start:].T
    t = jnp.pad(t, ((0, 0), (0, 128 - (n - start))))
    return t.reshape(8, 8, 128)


def kernel(user_ids, item_ids, user_factors, item_factors, user_bias, item_bias):
    uids = user_ids.astype(jnp.int32)
    iids = item_ids.astype(jnp.int32)
    uids2 = uids.reshape(_B // _CHUNK, _CHUNK)
    iids2 = iids.reshape(_B // _CHUNK, _CHUNK)
    uf4 = user_factors.T.reshape(8, 8, _NU)
    if4 = item_factors.T.reshape(8, 8, _NI)
    utail = _tail3(user_factors, _NU)
    itail = _tail3(item_factors, _NI)
    ub = user_bias.reshape(-1)
    ib = item_bias.reshape(-1)
    p, q = _k1(uids, iids, uf4, if4, utail, itail)
    return _k2(uids2, iids2, p, q, ub, ib)


# speedup vs baseline: 2.2983x; 1.0203x over previous
import jax
import jax.numpy as jnp
from jax import lax
from jax.experimental import pallas as pl
from jax.experimental.pallas import tpu as pltpu
from jax.experimental.pallas import tpu_sc as plsc

_B = 16384
_F = 64
_MEAN = 3.5
_NC = 2
_NS = 16
_NW = _NC * _NS
_BPW = _B // _NW
_CHUNK = 128
_NCHUNK = _BPW // _CHUNK
_NU = 1000000
_NI = 100000
_UCOLS = (_NU + 127) // 128
_ICOLS = (_NI + 127) // 128
_UPW = (_UCOLS + _NW - 1) // _NW
_IPW = (_ICOLS + _NW - 1) // _NW
_PMAX = 128
_HBUF = 272
_PROWS = _B + _NW


def _scan_table(tab_ref, tail_ref, ids_ref, out_ref, wid, ncols, perw,
                mu, mb, musort, mbsort, hist, offs, cur,
                slab0, slab1, slab2, slab3, pstage, bstage,
                s0, s1, s2, s3, psem):
    lane0 = lax.iota(jnp.int32, 16) == 0
    c0 = wid * perw
    c1 = jnp.minimum(c0 + perw, ncols)
    c0 = jnp.minimum(c0, ncols)

    pltpu.sync_copy(ids_ref, musort.at[pl.ds(0, _B)])

    def filt(k4, cnt):
        for e in range(4):
            k = k4 * 4 + e
            uvec = musort[pl.ds(k * 16, 16)]
            bvec = k * 16 + lax.iota(jnp.int32, 16)
            tc = uvec >> 7
            m = (tc >= c0) & (tc < c1)
            plsc.store_compressed(mu.at[pl.ds(cnt, 16)], uvec, mask=m)
            plsc.store_compressed(mb.at[pl.ds(cnt, 16)], bvec, mask=m)
            cnt = cnt + plsc.all_reduce_population_count(m)[0]
        return cnt
    kw = lax.fori_loop(0, _B // 64, filt, jnp.int32(0))

    for r in range(_HBUF // 16):
        hist[pl.ds(r * 16, 16)] = jnp.zeros((16,), jnp.int32)

    ones = jnp.ones((16,), jnp.int32)
    lanes16 = lax.iota(jnp.int32, 16)

    def hpass(k, carry):
        uvec = mu[pl.ds(k * 16, 16)]
        m = (k * 16 + lanes16) < kw
        rel = (uvec >> 7) - c0
        rel = jnp.where(m, rel, _HBUF - 1)
        plsc.addupdate_scatter(hist, [rel], ones, mask=m)
        return carry
    lax.fori_loop(0, (kw + 15) // 16, hpass, 0)

    def pref(r, carry):
        v = hist[pl.ds(r * 16, 16)]
        cs = plsc.cumsum(v)
        ex = cs - v + carry
        offs[pl.ds(r * 16, 16)] = ex
        cur[pl.ds(r * 16, 16)] = ex
        return carry + cs[15]
    lax.fori_loop(0, _HBUF // 16, pref, jnp.int32(0))

    def place(n, carry):
        u = mu[pl.ds(n, 16)][0]
        b = mb[pl.ds(n, 16)][0]
        rel = (u >> 7) - c0
        pos = cur[pl.ds(rel, 16)][0]
        plsc.store_scatter(musort, [jnp.full((16,), pos, jnp.int32)],
                           jnp.full((16,), u, jnp.int32), mask=lane0)
        plsc.store_scatter(mbsort, [jnp.full((16,), pos, jnp.int32)],
                           jnp.full((16,), b, jnp.int32), mask=lane0)
        plsc.store_scatter(cur, [jnp.full((16,), rel, jnp.int32)],
                           jnp.full((16,), pos + 1, jnp.int32), mask=lane0)
        return carry
    lax.fori_loop(0, kw, place, 0)

    dummy = _B + wid
    for r in range(_PMAX // 16):
        bstage[pl.ds(r * 16, 16)] = jnp.full((16,), dummy, jnp.int32)

    def fire(c, buf, sem):
        @pl.when(c < ncols - 1)
        def _():
            off = pl.multiple_of(c * 128, 128)
            pltpu.async_copy(tab_ref.at[:, :, pl.ds(off, 128)], buf, sem)

        @pl.when(c >= ncols - 1)
        def _():
            pltpu.async_copy(tail_ref, buf, sem)

    def process(c, buf, state):
        rel = c - c0
        lo = offs[pl.ds(rel, 16)][0]
        hi = offs[pl.ds(rel + 1, 16)][0]

        def one(n, pc2):
            u = musort[pl.ds(lo + n, 16)][0]
            b = mbsort[pl.ds(lo + n, 16)][0]
            col = u & 127
            for k in range(_F // 16):
                fvec = k * 16 + lax.iota(jnp.int32, 16)
                a = fvec >> 3
                s = fvec & 7
                cs = jnp.full((16,), col, jnp.int32)
                vals = plsc.load_gather(buf, [a, s, cs])
                pstage[pc2, pl.ds(k * 16, 16)] = vals
            plsc.store_scatter(
                bstage, [jnp.full((16,), pc2, jnp.int32)],
                jnp.full((16,), b, jnp.int32), mask=lane0)
            pc2 = pc2 + 1

            @pl.when(pc2 >= _PMAX)
            def _():
                pltpu.async_copy(pstage, out_ref.at[bstage], psem).wait()
                for r in range(_PMAX // 16):
                    bstage[pl.ds(r * 16, 16)] = jnp.full(
                        (16,), dummy, jnp.int32)
            return jnp.where(pc2 >= _PMAX, 0, pc2)
        return lax.fori_loop(0, hi - lo, one, state)

    n = c1 - c0
    slabs = (slab0, slab1, slab2, slab3)
    sems = (s0, s1, s2, s3)
    for j in range(4):
        @pl.when(c0 + j < c1)
        def _(j=j):
            fire(c0 + j, slabs[j], sems[j])

    def ring(t, pcnt):
        for j in range(4):
            c = c0 + 4 * t + j

            def go(pc, j=j, c=c):
                pltpu.make_async_copy(
                    tab_ref.at[:, :, pl.ds(0, 128)], slabs[j], sems[j]).wait()
                pc = process(c, slabs[j], pc)

                @pl.when(c + 4 < c1)
                def _():
                    fire(c + 4, slabs[j], sems[j])
                return pc
            pcnt = lax.cond(c < c1, go, lambda x: x, pcnt)
        return pcnt

    pcnt = lax.fori_loop(0, (n + 3) // 4, ring, jnp.int32(0))

    @pl.when(pcnt > 0)
    def _():
        pltpu.async_copy(pstage, out_ref.at[bstage], psem).wait()


def _k1_body(uids_ref, iids_ref, uf_ref, if_ref, utail_ref, itail_ref,
             p_out, q_out,
             mu, mb, musort, mbsort, hist, offs, cur,
             slab0, slab1, slab2, slab3, pstage, bstage,
             s0, s1, s2, s3, psem):
    wid = lax.axis_index("s") * _NC + lax.axis_index("c")
    _scan_table(uf_ref, utail_ref, uids_ref, p_out, wid, _UCOLS, _UPW,
                mu, mb, musort, mbsort, hist, offs, cur,
                slab0, slab1, slab2, slab3, pstage, bstage,
                s0, s1, s2, s3, psem)
    _scan_table(if_ref, itail_ref, iids_ref, q_out, wid, _ICOLS, _IPW,
                mu, mb, musort, mbsort, hist, offs, cur,
                slab0, slab1, slab2, slab3, pstage, bstage,
                s0, s1, s2, s3, psem)


@jax.jit
def _k1(uids, iids, uf4, if4, utail, itail):
    mesh = plsc.VectorSubcoreMesh(core_axis_name="c", subcore_axis_name="s")
    return pl.kernel(
        _k1_body,
        out_type=(
            jax.ShapeDtypeStruct((_PROWS, 128), jnp.float32),
            jax.ShapeDtypeStruct((_PROWS, 128), jnp.float32),
        ),
        mesh=mesh,
        compiler_params=pltpu.CompilerParams(
            needs_layout_passes=False, use_tc_tiling_on_sc=True),
        scratch_types=[
            pltpu.VMEM((_B + 32,), jnp.int32),
            pltpu.VMEM((_B + 32,), jnp.int32),
            pltpu.VMEM((_B + 32,), jnp.int32),
            pltpu.VMEM((_B + 32,), jnp.int32),
            pltpu.VMEM((_HBUF,), jnp.int32),
            pltpu.VMEM((_HBUF + 16,), jnp.int32),
            pltpu.VMEM((_HBUF + 16,), jnp.int32),
            pltpu.VMEM((8, 8, 128), jnp.float32),
            pltpu.VMEM((8, 8, 128), jnp.float32),
            pltpu.VMEM((8, 8, 128), jnp.float32),
            pltpu.VMEM((8, 8, 128), jnp.float32),
            pltpu.VMEM((_PMAX, 128), jnp.float32),
            pltpu.VMEM((_PMAX,), jnp.int32),
            pltpu.SemaphoreType.DMA,
            pltpu.SemaphoreType.DMA,
            pltpu.SemaphoreType.DMA,
            pltpu.SemaphoreType.DMA,
            pltpu.SemaphoreType.DMA,
        ],
    )(uids, iids, uf4, if4, utail, itail)


def _k2_body(uids_ref, iids_ref, p_ref, q_ref, ub_ref, ib_ref, out_ref,
             uid_v, iid_v, pv0, qv0, pv1, qv1, bu_v, bi_v, out_v, sem, sem2):
    pq_v = (pv0, qv0, pv1, qv1)
    wid = lax.axis_index("s") * _NC + lax.axis_index("c")

    pltpu.sync_copy(uids_ref.at[pl.ds(wid * _NCHUNK, _NCHUNK), :], uid_v)
    pltpu.sync_copy(iids_ref.at[pl.ds(wid * _NCHUNK, _NCHUNK), :], iid_v)

    bias_copies = []
    for j in range(_NCHUNK):
        sl = pl.ds(j * _CHUNK, _CHUNK)
        bias_copies.append(
            pltpu.async_copy(ub_ref.at[uid_v.at[j]], bu_v.at[sl], sem))
        bias_copies.append(
            pltpu.async_copy(ib_ref.at[iid_v.at[j]], bi_v.at[sl], sem))
    for cp in bias_copies:
        cp.wait()

    pq = []
    for h in range(2):
        base_row = wid * _BPW + h * 256
        pq.append(pltpu.async_copy(
            p_ref.at[pl.ds(base_row, 256), pl.ds(0, _F)], pq_v[2 * h], sem2))
        pq.append(pltpu.async_copy(
            q_ref.at[pl.ds(base_row, 256), pl.ds(0, _F)], pq_v[2 * h + 1], sem2))

    for h in range(2):
        pq[2 * h].wait()
        pq[2 * h + 1].wait()
        p_v = pq_v[2 * h]
        q_v = pq_v[2 * h + 1]

        def chunk16(c, carry):
            rows = c * 16 + lax.iota(jnp.int32, 16)
            gbase = h * 256 + c * 16
            acc = bu_v[pl.ds(gbase, 16)] + bi_v[pl.ds(gbase, 16)] + _MEAN
            for f in range(_F):
                fs = jnp.full((16,), f, jnp.int32)
                pv = plsc.load_gather(p_v, [rows, fs])
                qv = plsc.load_gather(q_v, [rows, fs])
                acc = acc + pv * qv
            out_v[pl.ds(gbase, 16)] = acc
            return carry
        lax.fori_loop(0, 16, chunk16, 0)

    pltpu.sync_copy(out_v, out_ref.at[pl.ds(wid * _BPW, _BPW)])


@jax.jit
def _k2(uids2, iids2, p, q, ub, ib):
    mesh = plsc.VectorSubcoreMesh(core_axis_name="c", subcore_axis_name="s")
    return pl.kernel(
        _k2_body,
        out_type=jax.ShapeDtypeStruct((_B,), jnp.float32),
        mesh=mesh,
        compiler_params=pltpu.CompilerParams(
            needs_layout_passes=False, use_tc_tiling_on_sc=False),
        scratch_types=[
            pltpu.VMEM((_NCHUNK, _CHUNK), jnp.int32),
            pltpu.VMEM((_NCHUNK, _CHUNK), jnp.int32),
            pltpu.VMEM((256, _F), jnp.float32),
            pltpu.VMEM((256, _F), jnp.float32),
            pltpu.VMEM((256, _F), jnp.float32),
            pltpu.VMEM((256, _F), jnp.float32),
            pltpu.VMEM((_BPW,), jnp.float32),
            pltpu.VMEM((_BPW,), jnp.float32),
            pltpu.VMEM((_BPW,), jnp.float32),
            pltpu.SemaphoreType.DMA,
            pltpu.SemaphoreType.DMA,
        ],
    )(uids2, iids2, p, q, ub, ib)


def _tail3(tab, n):
    start = (n // 128) * 128
    t = tab[start:].T
    t = jnp.pad(t, ((0, 0), (0, 128 - (n - start))))
    return t.reshape(8, 8, 128)


def kernel(user_ids, item_ids, user_factors, item_factors, user_bias, item_bias):
    uids = user_ids.astype(jnp.int32)
    iids = item_ids.astype(jnp.int32)
    uids2 = uids.reshape(_B // _CHUNK, _CHUNK)
    iids2 = iids.reshape(_B // _CHUNK, _CHUNK)
    uf4 = user_factors.T.reshape(8, 8, _NU)
    if4 = item_factors.T.reshape(8, 8, _NI)
    utail = _tail3(user_factors, _NU)
    itail = _tail3(item_factors, _NI)
    ub = user_bias.reshape(-1)
    ib = item_bias.reshape(-1)
    p, q = _k1(uids, iids, uf4, if4, utail, itail)
    return _k2(uids2, iids2, p, q, ub, ib)
